# Initial kernel scaffold; baseline (speedup 1.0000x reference)
import jax
import jax.numpy as jnp
from jax.experimental import pallas as pl


def kernel(x, w1, b1, w2, b2):
    raise NotImplementedError("write your pallas kernel here")


# speedup vs baseline: 1.3949x; 1.3949x over previous
import functools

import jax
import jax.numpy as jnp
from jax.experimental import pallas as pl
from jax.experimental.pallas import tpu as pltpu


def _gate_kernel(inv_hw, x_ref, w1t_ref, b1_ref, w2t_ref, b2_ref, o_ref):
    x = x_ref[...]
    s = jnp.sum(x, axis=-1, keepdims=True)
    m = jnp.max(x, axis=-1, keepdims=True)
    pooled = jnp.concatenate([s * inv_hw, m], axis=1)
    hidden = jnp.maximum(
        jnp.dot(w1t_ref[...], pooled,
                preferred_element_type=jnp.float32) + b1_ref[...], 0.0)
    att = jnp.dot(w2t_ref[...], hidden,
                  preferred_element_type=jnp.float32) + b2_ref[...]
    scale = jax.nn.sigmoid(att[:, 0:1] + att[:, 1:2])
    o_ref[...] = x * scale


def kernel(x, w1, b1, w2, b2):
    B, C, H, W = x.shape
    HW = H * W
    R = w1.shape[1]

    x2 = x.reshape(B * C, HW)
    w1t = w1.T
    w2t = w2.T
    b1c = b1.reshape(R, 1)
    b2c = b2.reshape(C, 1)

    out = pl.pallas_call(
        functools.partial(_gate_kernel, 1.0 / float(HW)),
        out_shape=jax.ShapeDtypeStruct((B * C, HW), x.dtype),
        grid=(B,),
        in_specs=[pl.BlockSpec((C, HW), lambda b: (b, 0)),
                  pl.BlockSpec((R, C), lambda b: (0, 0)),
                  pl.BlockSpec((R, 1), lambda b: (0, 0)),
                  pl.BlockSpec((C, R), lambda b: (0, 0)),
                  pl.BlockSpec((C, 1), lambda b: (0, 0))],
        out_specs=pl.BlockSpec((C, HW), lambda b: (b, 0)),
        compiler_params=pltpu.CompilerParams(
            dimension_semantics=("parallel",)),
    )(x2, w1t, b1c, w2t, b2c)

    return out.reshape(B, C, H, W)


# speedup vs baseline: 2.5091x; 1.7988x over previous
import functools

import jax
import jax.numpy as jnp
from jax.experimental import pallas as pl
from jax.experimental.pallas import tpu as pltpu


def _gate_kernel(inv_hw, x_ref, w1_ref, b1_ref, w2_ref, b2_ref, o_ref):
    x = x_ref[...]
    s = jnp.sum(x, axis=(1, 2))
    m = jnp.max(x, axis=(1, 2))
    pooled = jnp.stack([s * inv_hw, m], axis=0)
    hidden = jnp.maximum(
        jnp.dot(pooled, w1_ref[...],
                preferred_element_type=jnp.float32) + b1_ref[...], 0.0)
    att = jnp.dot(hidden, w2_ref[...],
                  preferred_element_type=jnp.float32) + b2_ref[...]
    scale = jax.nn.sigmoid(att[0:1, :] + att[1:2, :])
    o_ref[...] = x * scale.reshape(x.shape[0], 1, 1)


def kernel(x, w1, b1, w2, b2):
    B, C, H, W = x.shape
    R = w1.shape[1]

    x3 = x.reshape(B * C, H, W)
    b1r = b1.reshape(1, R)
    b2r = b2.reshape(1, C)

    out = pl.pallas_call(
        functools.partial(_gate_kernel, 1.0 / float(H * W)),
        out_shape=jax.ShapeDtypeStruct((B * C, H, W), x.dtype),
        grid=(B,),
        in_specs=[pl.BlockSpec((C, H, W), lambda b: (b, 0, 0)),
                  pl.BlockSpec((C, R), lambda b: (0, 0)),
                  pl.BlockSpec((1, R), lambda b: (0, 0)),
                  pl.BlockSpec((R, C), lambda b: (0, 0)),
                  pl.BlockSpec((1, C), lambda b: (0, 0))],
        out_specs=pl.BlockSpec((C, H, W), lambda b: (b, 0, 0)),
        compiler_params=pltpu.CompilerParams(
            dimension_semantics=("parallel",)),
    )(x3, w1, b1r, w2, b2r)

    return out.reshape(B, C, H, W)
